# Initial kernel scaffold; baseline (speedup 1.0000x reference)
import functools

import jax
import jax.numpy as jnp
from jax import lax
from jax.experimental import pallas as pl
from jax.experimental.pallas import tpu as pltpu
from jax.experimental.pallas import tpu_sc as plsc

N_NODES = 10000
D_FEAT = 128
N_EDGES = 320000

NC = 2
NS = 16
NW = NC * NS
CHUNK = 128
CPW = 80
EPW = CHUNK * CPW
EPAD = NW * EPW
NPAD = 10240
RPW = NPAD // NS


def _sc_spmm(x, src3, dst3, zeros_blk):
    mesh = plsc.VectorSubcoreMesh(core_axis_name="c", subcore_axis_name="s")

    @functools.partial(
        pl.kernel,
        out_type=jax.ShapeDtypeStruct((NC, NPAD, D_FEAT), jnp.float32),
        mesh=mesh,
        scratch_types=[
            pltpu.VMEM((CPW, CHUNK), jnp.int32),
            pltpu.VMEM((CPW, CHUNK), jnp.int32),
            pltpu.VMEM((CHUNK, D_FEAT), jnp.float32),
            pltpu.VMEM((CHUNK, D_FEAT), jnp.float32),
            pltpu.VMEM_SHARED((NPAD, D_FEAT), jnp.float32),
            pltpu.SemaphoreType.DMA,
            pltpu.SemaphoreType.DMA,
        ],
    )
    def spmm_kernel(x_hbm, src_hbm, dst_hbm, z_hbm, part_hbm,
                    src_v, dst_v, rows0, rows1, acc, sem0, sem1):
        c = lax.axis_index("c")
        s = lax.axis_index("s")
        wid = c * NS + s

        pltpu.sync_copy(src_hbm.at[wid], src_v)
        pltpu.sync_copy(dst_hbm.at[wid], dst_v)

        pltpu.sync_copy(z_hbm, rows0)
        for k in range(RPW // CHUNK):
            pltpu.sync_copy(rows0, acc.at[pl.ds(s * RPW + k * CHUNK, CHUNK)])
        plsc.subcore_barrier()

        pltpu.async_copy(x_hbm.at[src_v.at[0]], rows0, sem0)
        pltpu.async_copy(x_hbm.at[src_v.at[1]], rows1, sem1)

        def body(i, carry):
            j = 2 * i
            pltpu.make_async_copy(x_hbm.at[src_v.at[j]], rows0, sem0).wait()
            pltpu.sync_copy(rows0, acc.at[dst_v.at[j]], add=True)

            @pl.when(j + 2 < CPW)
            def _():
                pltpu.async_copy(x_hbm.at[src_v.at[j + 2]], rows0, sem0)

            pltpu.make_async_copy(x_hbm.at[src_v.at[j + 1]], rows1, sem1).wait()
            pltpu.sync_copy(rows1, acc.at[dst_v.at[j + 1]], add=True)

            @pl.when(j + 3 < CPW)
            def _():
                pltpu.async_copy(x_hbm.at[src_v.at[j + 3]], rows1, sem1)

            return carry
        lax.fori_loop(0, CPW // 2, body, 0)
        plsc.subcore_barrier()

        pltpu.sync_copy(acc.at[pl.ds(s * RPW, RPW)],
                        part_hbm.at[c, pl.ds(s * RPW, RPW)])

    return spmm_kernel(x, src3, dst3, zeros_blk)


def _tc_add(part):
    def add_kernel(a_ref, b_ref, o_ref):
        o_ref[...] = a_ref[...] + b_ref[...]

    return pl.pallas_call(
        add_kernel,
        grid=(10,),
        in_specs=[
            pl.BlockSpec((None, 1000, D_FEAT), lambda i: (0, i, 0)),
            pl.BlockSpec((None, 1000, D_FEAT), lambda i: (1, i, 0)),
        ],
        out_specs=pl.BlockSpec((1000, D_FEAT), lambda i: (i, 0)),
        out_shape=jax.ShapeDtypeStruct((N_NODES, D_FEAT), jnp.float32),
    )(part, part)


def kernel(x, edge_index):
    src = edge_index[0].astype(jnp.int32)
    dst = edge_index[1].astype(jnp.int32)
    pad = EPAD - N_EDGES
    src = jnp.concatenate([src, jnp.zeros((pad,), jnp.int32)])
    dst = jnp.concatenate([dst, jnp.full((pad,), N_NODES, jnp.int32)])
    src3 = src.reshape(NW, CPW, CHUNK)
    dst3 = dst.reshape(NW, CPW, CHUNK)
    zeros_blk = jnp.zeros((CHUNK, D_FEAT), jnp.float32)
    part = _sc_spmm(x, src3, dst3, zeros_blk)
    return _tc_add(part)


# speedup vs baseline: 3.4146x; 3.4146x over previous
import functools

import jax
import jax.numpy as jnp
from jax import lax
from jax.experimental import pallas as pl
from jax.experimental.pallas import tpu as pltpu
from jax.experimental.pallas import tpu_sc as plsc

N_NODES = 10000
D_FEAT = 128
N_EDGES = 320000

NC = 2
NS = 16
NW = NC * NS
CHUNK = 128
CPW = 80
EPW = CHUNK * CPW
EPAD = NW * EPW
NPAD = 10240
RPW = NPAD // NS


def _sc_spmm(x, idx4, zeros_blk):
    mesh = plsc.VectorSubcoreMesh(core_axis_name="c", subcore_axis_name="s")

    @functools.partial(
        pl.kernel,
        out_type=jax.ShapeDtypeStruct((NC, NPAD, D_FEAT), jnp.float32),
        mesh=mesh,
        scratch_types=[
            pltpu.VMEM((2, 2, CHUNK), jnp.int32),
            pltpu.VMEM((CHUNK, D_FEAT), jnp.float32),
            pltpu.VMEM((CHUNK, D_FEAT), jnp.float32),
            pltpu.VMEM_SHARED((NPAD, D_FEAT), jnp.float32),
            pltpu.SemaphoreType.DMA,
            pltpu.SemaphoreType.DMA,
            pltpu.SemaphoreType.DMA,
            pltpu.SemaphoreType.DMA,
        ],
    )
    def spmm_kernel(x_hbm, idx_hbm, z_hbm, part_hbm,
                    idx_v, rows0, rows1, acc, semi0, semi1, semg0, semg1):
        c = lax.axis_index("c")
        s = lax.axis_index("s")
        wid = c * NS + s
        rows = (rows0, rows1)
        semi = (semi0, semi1)
        semg = (semg0, semg1)

        pltpu.sync_copy(z_hbm, rows0)
        for k in range(RPW // CHUNK):
            pltpu.sync_copy(rows0, acc.at[pl.ds(s * RPW + k * CHUNK, CHUNK)])
        plsc.subcore_barrier()

        for b in range(2):
            pltpu.async_copy(idx_hbm.at[wid, b], idx_v.at[b], semi[b])

        def body(i, carry):
            j = 2 * i
            for b in range(2):
                pltpu.make_async_copy(idx_hbm.at[wid, j + b], idx_v.at[b],
                                      semi[b]).wait()
                pltpu.async_copy(x_hbm.at[idx_v.at[b, 0]], rows[b], semg[b])
            for b in range(2):
                pltpu.make_async_copy(x_hbm.at[idx_v.at[b, 0]], rows[b],
                                      semg[b]).wait()
                pltpu.sync_copy(rows[b], acc.at[idx_v.at[b, 1]], add=True)

                @pl.when(j + b + 2 < CPW)
                def _():
                    pltpu.async_copy(idx_hbm.at[wid, j + b + 2], idx_v.at[b],
                                     semi[b])
            return carry
        lax.fori_loop(0, CPW // 2, body, 0)
        plsc.subcore_barrier()

        pltpu.sync_copy(acc.at[pl.ds(s * RPW, RPW)],
                        part_hbm.at[c, pl.ds(s * RPW, RPW)])

    return spmm_kernel(x, idx4, zeros_blk)


def _tc_add(part):
    def add_kernel(a_ref, b_ref, o_ref):
        o_ref[...] = a_ref[...] + b_ref[...]

    return pl.pallas_call(
        add_kernel,
        grid=(10,),
        in_specs=[
            pl.BlockSpec((None, 1000, D_FEAT), lambda i: (0, i, 0)),
            pl.BlockSpec((None, 1000, D_FEAT), lambda i: (1, i, 0)),
        ],
        out_specs=pl.BlockSpec((1000, D_FEAT), lambda i: (i, 0)),
        out_shape=jax.ShapeDtypeStruct((N_NODES, D_FEAT), jnp.float32),
    )(part, part)


def kernel(x, edge_index):
    src = edge_index[0].astype(jnp.int32)
    dst = edge_index[1].astype(jnp.int32)
    pad = EPAD - N_EDGES
    src = jnp.concatenate([src, jnp.zeros((pad,), jnp.int32)])
    dst = jnp.concatenate([dst, jnp.full((pad,), N_NODES, jnp.int32)])
    idx4 = jnp.stack(
        [src.reshape(NW, CPW, CHUNK), dst.reshape(NW, CPW, CHUNK)], axis=2)
    zeros_blk = jnp.zeros((CHUNK, D_FEAT), jnp.float32)
    part = _sc_spmm(x, idx4, zeros_blk)
    return _tc_add(part)


# speedup vs baseline: 3.4207x; 1.0018x over previous
import functools

import jax
import jax.numpy as jnp
from jax import lax
from jax.experimental import pallas as pl
from jax.experimental.pallas import tpu as pltpu
from jax.experimental.pallas import tpu_sc as plsc

N_NODES = 10000
D_FEAT = 128
N_EDGES = 320000

NC = 2
NS = 16
NW = NC * NS
CHUNK = 128
CPW = 80
EPW = CHUNK * CPW
EPAD = NW * EPW
NPAD = 10240
RPW = NPAD // NS


def _sc_spmm(x, idx4, zeros_blk):
    mesh = plsc.VectorSubcoreMesh(core_axis_name="c", subcore_axis_name="s")

    @functools.partial(
        pl.kernel,
        out_type=jax.ShapeDtypeStruct((NC, NPAD, D_FEAT), jnp.float32),
        mesh=mesh,
        scratch_types=[
            pltpu.VMEM((2, 2, CHUNK), jnp.int32),
            pltpu.VMEM((CHUNK, D_FEAT), jnp.float32),
            pltpu.VMEM((CHUNK, D_FEAT), jnp.float32),
            pltpu.VMEM_SHARED((NPAD, D_FEAT), jnp.float32),
            pltpu.SemaphoreType.DMA,
            pltpu.SemaphoreType.DMA,
            pltpu.SemaphoreType.DMA,
            pltpu.SemaphoreType.DMA,
        ],
    )
    def spmm_kernel(x_hbm, idx_hbm, z_hbm, part_hbm,
                    idx_v, rows0, rows1, acc, semi0, semi1, semg0, semg1):
        c = lax.axis_index("c")
        s = lax.axis_index("s")
        wid = c * NS + s
        rows = (rows0, rows1)
        semi = (semi0, semi1)
        semg = (semg0, semg1)

        pltpu.sync_copy(z_hbm, rows0)
        for k in range(RPW // CHUNK):
            pltpu.sync_copy(rows0, acc.at[pl.ds(s * RPW + k * CHUNK, CHUNK)])
        plsc.subcore_barrier()

        for b in range(2):
            pltpu.async_copy(idx_hbm.at[wid, b], idx_v.at[b], semi[b])

        def body(i, carry):
            j = 2 * i
            for b in range(2):
                pltpu.make_async_copy(idx_hbm.at[wid, j + b], idx_v.at[b],
                                      semi[b]).wait()
                pltpu.async_copy(x_hbm.at[idx_v.at[b, 0]], rows[b], semg[b])
            for b in range(2):
                pltpu.make_async_copy(x_hbm.at[idx_v.at[b, 0]], rows[b],
                                      semg[b]).wait()
                pltpu.sync_copy(rows[b], acc.at[idx_v.at[b, 1]], add=True)

                @pl.when(j + b + 2 < CPW)
                def _():
                    pltpu.async_copy(idx_hbm.at[wid, j + b + 2], idx_v.at[b],
                                     semi[b])
            return carry
        lax.fori_loop(0, CPW // 2, body, 0)
        plsc.subcore_barrier()

        pltpu.sync_copy(acc.at[pl.ds(s * RPW, RPW)],
                        part_hbm.at[c, pl.ds(s * RPW, RPW)])

    return spmm_kernel(x, idx4, zeros_blk)


def _tc_add(part):
    def add_kernel(a_ref, b_ref, o_ref):
        o_ref[...] = a_ref[...] + b_ref[...]

    return pl.pallas_call(
        add_kernel,
        grid=(10,),
        in_specs=[
            pl.BlockSpec((None, 1000, D_FEAT), lambda i: (0, i, 0)),
            pl.BlockSpec((None, 1000, D_FEAT), lambda i: (1, i, 0)),
        ],
        out_specs=pl.BlockSpec((1000, D_FEAT), lambda i: (i, 0)),
        out_shape=jax.ShapeDtypeStruct((N_NODES, D_FEAT), jnp.float32),
    )(part, part)


def kernel(x, edge_index):
    src = edge_index[0].astype(jnp.int32)
    dst = edge_index[1].astype(jnp.int32)
    pad = EPAD - N_EDGES
    src = jnp.concatenate([src, jnp.zeros((pad,), jnp.int32)])
    pad_dst = N_NODES + (jnp.arange(pad, dtype=jnp.int32) % (NPAD - N_NODES))
    dst = jnp.concatenate([dst, pad_dst])
    idx4 = jnp.stack(
        [src.reshape(NW, CPW, CHUNK), dst.reshape(NW, CPW, CHUNK)], axis=2)
    zeros_blk = jnp.zeros((CHUNK, D_FEAT), jnp.float32)
    part = _sc_spmm(x, idx4, zeros_blk)
    return _tc_add(part)


# speedup vs baseline: 10.5814x; 3.0933x over previous
import functools

import jax
import jax.numpy as jnp
from jax import lax
from jax.experimental import pallas as pl
from jax.experimental.pallas import tpu as pltpu
from jax.experimental.pallas import tpu_sc as plsc

N_NODES = 10000
D_FEAT = 128
N_EDGES = 320000

NC = 2
NS = 16
NW = NC * NS
CHUNK = 128
CPW = 80
EPW = CHUNK * CPW
EPAD = NW * EPW
NPAD = 10240
RPW = NPAD // NS


def _sc_spmm(x, idx4, zeros_blk):
    mesh = plsc.VectorSubcoreMesh(core_axis_name="c", subcore_axis_name="s")

    @functools.partial(
        pl.kernel,
        out_type=jax.ShapeDtypeStruct((NC, NPAD, D_FEAT), jnp.float32),
        mesh=mesh,
        scratch_types=[
            pltpu.VMEM((2, 2, CHUNK), jnp.int32),
            pltpu.VMEM((CHUNK, D_FEAT), jnp.float32),
            pltpu.VMEM((CHUNK, D_FEAT), jnp.float32),
            pltpu.VMEM_SHARED((NPAD, D_FEAT), jnp.float32),
            pltpu.SemaphoreType.DMA,
            pltpu.SemaphoreType.DMA,
            pltpu.SemaphoreType.DMA,
            pltpu.SemaphoreType.DMA,
        ],
    )
    def spmm_kernel(x_hbm, idx_hbm, z_hbm, part_hbm,
                    idx_v, rows0, rows1, acc, semi0, semi1, semg0, semg1):
        c = lax.axis_index("c")
        s = lax.axis_index("s")
        wid = c * NS + s
        rows = (rows0, rows1)
        semi = (semi0, semi1)
        semg = (semg0, semg1)

        pltpu.sync_copy(z_hbm, rows0)
        for k in range(RPW // CHUNK):
            pltpu.sync_copy(rows0, acc.at[pl.ds(s * RPW + k * CHUNK, CHUNK)])
        plsc.subcore_barrier()

        for b in range(2):
            pltpu.async_copy(idx_hbm.at[wid, b], idx_v.at[b], semi[b])

        def body(i, carry):
            j = 2 * i
            for b in range(2):
                pltpu.make_async_copy(idx_hbm.at[wid, j + b], idx_v.at[b],
                                      semi[b]).wait()
                pltpu.async_copy(x_hbm.at[idx_v.at[b, 0]], rows[b], semg[b])
            for b in range(2):
                pltpu.make_async_copy(x_hbm.at[idx_v.at[b, 0]], rows[b],
                                      semg[b]).wait()
                pltpu.sync_copy(rows[b], acc.at[idx_v.at[b, 1]], add=True)

                @pl.when(j + b + 2 < CPW)
                def _():
                    pltpu.async_copy(idx_hbm.at[wid, j + b + 2], idx_v.at[b],
                                     semi[b])
            return carry
        lax.fori_loop(0, CPW // 2, body, 0)
        plsc.subcore_barrier()

        pltpu.sync_copy(acc.at[pl.ds(s * RPW, RPW)],
                        part_hbm.at[c, pl.ds(s * RPW, RPW)])

    return spmm_kernel(x, idx4, zeros_blk)


def _tc_add(part):
    def add_kernel(a_ref, b_ref, o_ref):
        o_ref[...] = a_ref[...] + b_ref[...]

    return pl.pallas_call(
        add_kernel,
        grid=(10,),
        in_specs=[
            pl.BlockSpec((None, 1000, D_FEAT), lambda i: (0, i, 0)),
            pl.BlockSpec((None, 1000, D_FEAT), lambda i: (1, i, 0)),
        ],
        out_specs=pl.BlockSpec((1000, D_FEAT), lambda i: (i, 0)),
        out_shape=jax.ShapeDtypeStruct((N_NODES, D_FEAT), jnp.float32),
    )(part, part)


def kernel(x, edge_index):
    src = edge_index[0].astype(jnp.int32)
    dst = edge_index[1].astype(jnp.int32)
    pad = EPAD - N_EDGES
    pad_src = jnp.arange(pad, dtype=jnp.int32) % N_NODES
    src = jnp.concatenate([src, pad_src])
    pad_dst = N_NODES + (jnp.arange(pad, dtype=jnp.int32) % (NPAD - N_NODES))
    dst = jnp.concatenate([dst, pad_dst])
    idx4 = jnp.stack(
        [src.reshape(NW, CPW, CHUNK), dst.reshape(NW, CPW, CHUNK)], axis=2)
    zeros_blk = jnp.zeros((CHUNK, D_FEAT), jnp.float32)
    part = _sc_spmm(x, idx4, zeros_blk)
    return _tc_add(part)


# speedup vs baseline: 11.1593x; 1.0546x over previous
import functools

import jax
import jax.numpy as jnp
from jax import lax
from jax.experimental import pallas as pl
from jax.experimental.pallas import tpu as pltpu
from jax.experimental.pallas import tpu_sc as plsc

N_NODES = 10000
D_FEAT = 128
N_EDGES = 320000

NC = 2
NS = 16
NW = NC * NS
CHUNK = 64
CPW = 160
NBUF = 4
NIDX = 8
EPW = CHUNK * CPW
EPAD = NW * EPW
NPAD = 10240
RPW = NPAD // NS


def _sc_spmm(x, idx4, zeros_blk):
    mesh = plsc.VectorSubcoreMesh(core_axis_name="c", subcore_axis_name="s")

    @functools.partial(
        pl.kernel,
        out_type=jax.ShapeDtypeStruct((NC, NPAD, D_FEAT), jnp.float32),
        mesh=mesh,
        scratch_types=(
            [pltpu.VMEM((NIDX, 2, CHUNK), jnp.int32)]
            + [pltpu.VMEM((CHUNK, D_FEAT), jnp.float32)] * NBUF
            + [pltpu.VMEM_SHARED((NPAD, D_FEAT), jnp.float32)]
            + [pltpu.SemaphoreType.DMA] * (NIDX + NBUF + NBUF)
        ),
    )
    def spmm_kernel(x_hbm, idx_hbm, z_hbm, part_hbm, idx_v, *rest):
        rows = rest[:NBUF]
        acc = rest[NBUF]
        semi = rest[NBUF + 1:NBUF + 1 + NIDX]
        semg = rest[NBUF + 1 + NIDX:NBUF + 1 + NIDX + NBUF]
        sems = rest[NBUF + 1 + NIDX + NBUF:]

        c = lax.axis_index("c")
        s = lax.axis_index("s")
        wid = c * NS + s

        def idx_start(j, b):
            pltpu.async_copy(idx_hbm.at[wid, j], idx_v.at[b], semi[b])

        def idx_wait(j, b):
            pltpu.make_async_copy(idx_hbm.at[wid, j], idx_v.at[b],
                                  semi[b]).wait()

        def gather_start(b, q):
            pltpu.async_copy(x_hbm.at[idx_v.at[b, 0]], rows[q], semg[q])

        def gather_wait(b, q):
            pltpu.make_async_copy(x_hbm.at[idx_v.at[b, 0]], rows[q],
                                  semg[q]).wait()

        def scat_start(b, q):
            pltpu.async_copy(rows[q], acc.at[idx_v.at[b, 1]], sems[q],
                             add=True)

        def scat_wait(b, q):
            pltpu.make_async_copy(rows[q], acc.at[idx_v.at[b, 1]],
                                  sems[q]).wait()

        pltpu.sync_copy(z_hbm, rows[0])
        for k in range(RPW // CHUNK):
            pltpu.sync_copy(rows[0], acc.at[pl.ds(s * RPW + k * CHUNK, CHUNK)])
        plsc.subcore_barrier()

        for t in range(5):
            idx_start(t, t)
        idx_wait(0, 0)
        gather_start(0, 0)

        def body(i, carry):
            base = 8 * i
            for u in range(8):
                j = base + u
                q = u % NBUF
                q1 = (u + 1) % NBUF
                b1 = (u + 1) % NIDX
                b5 = (u + 5) % NIDX
                bm3 = (u - 3) % NIDX

                @pl.when(j >= 3)
                def _():
                    scat_wait(bm3, q1)

                @pl.when(j + 5 < CPW)
                def _():
                    idx_start(j + 5, b5)

                @pl.when(j + 1 < CPW)
                def _():
                    idx_wait(j + 1, b1)
                    gather_start(b1, q1)

                gather_wait(u, q)
                scat_start(u, q)
            return carry
        lax.fori_loop(0, CPW // 8, body, 0)

        for t in (CPW - 3, CPW - 2, CPW - 1):
            scat_wait(t % NIDX, t % NBUF)
        plsc.subcore_barrier()

        pltpu.sync_copy(acc.at[pl.ds(s * RPW, RPW)],
                        part_hbm.at[c, pl.ds(s * RPW, RPW)])

    return spmm_kernel(x, idx4, zeros_blk)


def _tc_add(part):
    def add_kernel(a_ref, b_ref, o_ref):
        o_ref[...] = a_ref[...] + b_ref[...]

    return pl.pallas_call(
        add_kernel,
        grid=(10,),
        in_specs=[
            pl.BlockSpec((None, 1000, D_FEAT), lambda i: (0, i, 0)),
            pl.BlockSpec((None, 1000, D_FEAT), lambda i: (1, i, 0)),
        ],
        out_specs=pl.BlockSpec((1000, D_FEAT), lambda i: (i, 0)),
        out_shape=jax.ShapeDtypeStruct((N_NODES, D_FEAT), jnp.float32),
    )(part, part)


def kernel(x, edge_index):
    src = edge_index[0].astype(jnp.int32)
    dst = edge_index[1].astype(jnp.int32)
    pad = EPAD - N_EDGES
    pad_src = jnp.arange(pad, dtype=jnp.int32) % N_NODES
    src = jnp.concatenate([src, pad_src])
    pad_dst = N_NODES + (jnp.arange(pad, dtype=jnp.int32) % (NPAD - N_NODES))
    dst = jnp.concatenate([dst, pad_dst])
    idx4 = jnp.stack(
        [src.reshape(NW, CPW, CHUNK), dst.reshape(NW, CPW, CHUNK)], axis=2)
    zeros_blk = jnp.zeros((CHUNK, D_FEAT), jnp.float32)
    part = _sc_spmm(x, idx4, zeros_blk)
    return _tc_add(part)


# speedup vs baseline: 12.5169x; 1.1217x over previous
import functools

import jax
import jax.numpy as jnp
from jax import lax
from jax.experimental import pallas as pl
from jax.experimental.pallas import tpu as pltpu
from jax.experimental.pallas import tpu_sc as plsc

N_NODES = 10000
D_FEAT = 128
N_EDGES = 320000

NC = 2
NS = 16
NW = NC * NS
CHUNK = 120
CPW = 84
NBUF = 3
NIDX = 6
EPW = CHUNK * CPW
EPAD = NW * EPW
NPAD = 10112
RPW = NPAD // NS


def _sc_spmm(x, idx4, zeros_blk):
    mesh = plsc.VectorSubcoreMesh(core_axis_name="c", subcore_axis_name="s")

    @functools.partial(
        pl.kernel,
        out_type=jax.ShapeDtypeStruct((NC, NPAD, D_FEAT), jnp.float32),
        mesh=mesh,
        scratch_types=(
            [pltpu.VMEM((NIDX, 2, CHUNK), jnp.int32)]
            + [pltpu.VMEM((CHUNK, D_FEAT), jnp.float32)] * NBUF
            + [pltpu.VMEM_SHARED((NPAD, D_FEAT), jnp.float32)]
            + [pltpu.SemaphoreType.DMA] * (NIDX + NBUF + NBUF)
        ),
    )
    def spmm_kernel(x_hbm, idx_hbm, z_hbm, part_hbm, idx_v, *rest):
        rows = rest[:NBUF]
        acc = rest[NBUF]
        semi = rest[NBUF + 1:NBUF + 1 + NIDX]
        semg = rest[NBUF + 1 + NIDX:NBUF + 1 + NIDX + NBUF]
        sems = rest[NBUF + 1 + NIDX + NBUF:]

        c = lax.axis_index("c")
        s = lax.axis_index("s")
        wid = c * NS + s

        def idx_start(j, b):
            pltpu.async_copy(idx_hbm.at[wid, j], idx_v.at[b], semi[b])

        def idx_wait(j, b):
            pltpu.make_async_copy(idx_hbm.at[wid, j], idx_v.at[b],
                                  semi[b]).wait()

        def gather_start(b, q):
            pltpu.async_copy(x_hbm.at[idx_v.at[b, 0]], rows[q], semg[q])

        def gather_wait(b, q):
            pltpu.make_async_copy(x_hbm.at[idx_v.at[b, 0]], rows[q],
                                  semg[q]).wait()

        def scat_start(b, q):
            pltpu.async_copy(rows[q], acc.at[idx_v.at[b, 1]], sems[q],
                             add=True)

        def scat_wait(b, q):
            pltpu.make_async_copy(rows[q], acc.at[idx_v.at[b, 1]],
                                  sems[q]).wait()

        pltpu.sync_copy(z_hbm, rows[0])
        base0 = s * RPW
        for k in range(RPW // CHUNK):
            pltpu.sync_copy(rows[0], acc.at[pl.ds(base0 + k * CHUNK, CHUNK)])
        rem = RPW - (RPW // CHUNK) * CHUNK
        if rem:
            pltpu.sync_copy(rows[0].at[pl.ds(0, rem)],
                            acc.at[pl.ds(base0 + RPW - rem, rem)])
        plsc.subcore_barrier()

        for t in range(4):
            idx_start(t, t)
        idx_wait(0, 0)
        gather_start(0, 0)

        def body(i, carry):
            base = 6 * i
            for u in range(6):
                j = base + u
                q = u % NBUF
                q1 = (u + 1) % NBUF
                b1 = (u + 1) % NIDX
                b4 = (u + 4) % NIDX
                bm2 = (u - 2) % NIDX

                @pl.when(j >= 2)
                def _():
                    scat_wait(bm2, q1)

                @pl.when(j + 4 < CPW)
                def _():
                    idx_start(j + 4, b4)

                @pl.when(j + 1 < CPW)
                def _():
                    idx_wait(j + 1, b1)
                    gather_start(b1, q1)

                gather_wait(u, q)
                scat_start(u, q)
            return carry
        lax.fori_loop(0, CPW // 6, body, 0)

        for t in (CPW - 2, CPW - 1):
            scat_wait(t % NIDX, t % NBUF)
        plsc.subcore_barrier()

        pltpu.sync_copy(acc.at[pl.ds(s * RPW, RPW)],
                        part_hbm.at[c, pl.ds(s * RPW, RPW)])

    return spmm_kernel(x, idx4, zeros_blk)


def _tc_add(part):
    def add_kernel(a_ref, b_ref, o_ref):
        o_ref[...] = a_ref[...] + b_ref[...]

    return pl.pallas_call(
        add_kernel,
        grid=(10,),
        in_specs=[
            pl.BlockSpec((None, 1000, D_FEAT), lambda i: (0, i, 0)),
            pl.BlockSpec((None, 1000, D_FEAT), lambda i: (1, i, 0)),
        ],
        out_specs=pl.BlockSpec((1000, D_FEAT), lambda i: (i, 0)),
        out_shape=jax.ShapeDtypeStruct((N_NODES, D_FEAT), jnp.float32),
    )(part, part)


def kernel(x, edge_index):
    src = edge_index[0].astype(jnp.int32)
    dst = edge_index[1].astype(jnp.int32)
    pad = EPAD - N_EDGES
    pad_src = jnp.arange(pad, dtype=jnp.int32) % N_NODES
    src = jnp.concatenate([src, pad_src])
    pad_dst = N_NODES + (jnp.arange(pad, dtype=jnp.int32) % (NPAD - N_NODES))
    dst = jnp.concatenate([dst, pad_dst])
    idx4 = jnp.stack(
        [src.reshape(NW, CPW, CHUNK), dst.reshape(NW, CPW, CHUNK)], axis=2)
    zeros_blk = jnp.zeros((CHUNK, D_FEAT), jnp.float32)
    part = _sc_spmm(x, idx4, zeros_blk)
    return _tc_add(part)


# speedup vs baseline: 13.2235x; 1.0565x over previous
import functools

import jax
import jax.numpy as jnp
from jax import lax
from jax.experimental import pallas as pl
from jax.experimental.pallas import tpu as pltpu
from jax.experimental.pallas import tpu_sc as plsc

N_NODES = 10000
D_FEAT = 128
N_EDGES = 320000

NC = 2
NS = 16
NW = NC * NS
CHUNK = 120
CPW = 84
NBUF = 3
NIDX = 6
EPW = CHUNK * CPW
EPAD = NW * EPW
NPAD = 10112
RPW = NPAD // NS


def _sc_spmm(x, src3, dst3, zeros_blk):
    mesh = plsc.VectorSubcoreMesh(core_axis_name="c", subcore_axis_name="s")

    @functools.partial(
        pl.kernel,
        out_type=jax.ShapeDtypeStruct((NC, NPAD, D_FEAT), jnp.float32),
        mesh=mesh,
        scratch_types=(
            [pltpu.VMEM((NIDX, CHUNK), jnp.int32)] * 2
            + [pltpu.VMEM((CHUNK, D_FEAT), jnp.float32)] * NBUF
            + [pltpu.VMEM_SHARED((NPAD, D_FEAT), jnp.float32)]
            + [pltpu.SemaphoreType.DMA] * (2 * NIDX + NBUF + NBUF)
        ),
    )
    def spmm_kernel(x_hbm, src_hbm, dst_hbm, z_hbm, part_hbm, isrc, idst,
                    *rest):
        rows = rest[:NBUF]
        acc = rest[NBUF]
        semis = rest[NBUF + 1:NBUF + 1 + NIDX]
        semid = rest[NBUF + 1 + NIDX:NBUF + 1 + 2 * NIDX]
        semg = rest[NBUF + 1 + 2 * NIDX:NBUF + 1 + 2 * NIDX + NBUF]
        sems = rest[NBUF + 1 + 2 * NIDX + NBUF:]

        c = lax.axis_index("c")
        s = lax.axis_index("s")
        wid = c * NS + s

        def idx_start(j, b):
            pltpu.async_copy(src_hbm.at[wid, j], isrc.at[b], semis[b])
            pltpu.async_copy(dst_hbm.at[wid, j], idst.at[b], semid[b])

        def idx_wait(j, b):
            pltpu.make_async_copy(src_hbm.at[wid, j], isrc.at[b],
                                  semis[b]).wait()
            pltpu.make_async_copy(dst_hbm.at[wid, j], idst.at[b],
                                  semid[b]).wait()

        def gather_start(b, q):
            pltpu.async_copy(x_hbm.at[isrc.at[b]], rows[q], semg[q])

        def gather_wait(b, q):
            pltpu.make_async_copy(x_hbm.at[isrc.at[b]], rows[q],
                                  semg[q]).wait()

        def scat_start(b, q):
            pltpu.async_copy(rows[q], acc.at[idst.at[b]], sems[q],
                             add=True)

        def scat_wait(b, q):
            pltpu.make_async_copy(rows[q], acc.at[idst.at[b]],
                                  sems[q]).wait()

        pltpu.sync_copy(z_hbm, rows[0])
        base0 = s * RPW
        for k in range(RPW // CHUNK):
            pltpu.sync_copy(rows[0], acc.at[pl.ds(base0 + k * CHUNK, CHUNK)])
        rem = RPW - (RPW // CHUNK) * CHUNK
        if rem:
            pltpu.sync_copy(rows[0].at[pl.ds(0, rem)],
                            acc.at[pl.ds(base0 + RPW - rem, rem)])
        plsc.subcore_barrier()

        for t in range(4):
            idx_start(t, t)
        idx_wait(0, 0)
        gather_start(0, 0)

        def body(i, carry):
            base = 6 * i
            for u in range(6):
                j = base + u
                q = u % NBUF
                q1 = (u + 1) % NBUF
                b1 = (u + 1) % NIDX
                b4 = (u + 4) % NIDX
                bm2 = (u - 2) % NIDX

                @pl.when(j >= 2)
                def _():
                    scat_wait(bm2, q1)

                @pl.when(j + 4 < CPW)
                def _():
                    idx_start(j + 4, b4)

                @pl.when(j + 1 < CPW)
                def _():
                    idx_wait(j + 1, b1)
                    gather_start(b1, q1)

                gather_wait(u, q)
                scat_start(u, q)
            return carry
        lax.fori_loop(0, CPW // 6, body, 0)

        for t in (CPW - 2, CPW - 1):
            scat_wait(t % NIDX, t % NBUF)
        plsc.subcore_barrier()

        pltpu.sync_copy(acc.at[pl.ds(s * RPW, RPW)],
                        part_hbm.at[c, pl.ds(s * RPW, RPW)])

    return spmm_kernel(x, src3, dst3, zeros_blk)


def _tc_add(part):
    def add_kernel(a_ref, b_ref, o_ref):
        o_ref[...] = a_ref[...] + b_ref[...]

    return pl.pallas_call(
        add_kernel,
        grid=(10,),
        in_specs=[
            pl.BlockSpec((None, 1000, D_FEAT), lambda i: (0, i, 0)),
            pl.BlockSpec((None, 1000, D_FEAT), lambda i: (1, i, 0)),
        ],
        out_specs=pl.BlockSpec((1000, D_FEAT), lambda i: (i, 0)),
        out_shape=jax.ShapeDtypeStruct((N_NODES, D_FEAT), jnp.float32),
    )(part, part)


def kernel(x, edge_index):
    src = edge_index[0].astype(jnp.int32)
    dst = edge_index[1].astype(jnp.int32)
    pad = EPAD - N_EDGES
    pad_src = jnp.arange(pad, dtype=jnp.int32) % N_NODES
    src = jnp.concatenate([src, pad_src])
    pad_dst = N_NODES + (jnp.arange(pad, dtype=jnp.int32) % (NPAD - N_NODES))
    dst = jnp.concatenate([dst, pad_dst])
    src3 = src.reshape(NW, CPW, CHUNK)
    dst3 = dst.reshape(NW, CPW, CHUNK)
    zeros_blk = jnp.zeros((CHUNK, D_FEAT), jnp.float32)
    part = _sc_spmm(x, src3, dst3, zeros_blk)
    return _tc_add(part)


# speedup vs baseline: 14.1556x; 1.0705x over previous
import functools

import jax
import jax.numpy as jnp
from jax import lax
from jax.experimental import pallas as pl
from jax.experimental.pallas import tpu as pltpu
from jax.experimental.pallas import tpu_sc as plsc

N_NODES = 10000
D_FEAT = 128
N_EDGES = 320000

NC = 2
NS = 16
NW = NC * NS
CHUNK = 120
CPW = 84
CPWM = 83
EPT = N_EDGES // NW
TPAD = CPW * CHUNK - EPT
NBUF = 3
NIDX = 6
NPAD = 10112
RPW = NPAD // NS


def _sc_spmm(x, ei, tail, zeros_blk):
    mesh = plsc.VectorSubcoreMesh(core_axis_name="c", subcore_axis_name="s")

    @functools.partial(
        pl.kernel,
        out_type=jax.ShapeDtypeStruct((NC, NPAD, D_FEAT), jnp.float32),
        mesh=mesh,
        scratch_types=(
            [pltpu.VMEM((NIDX, CHUNK), jnp.int32)] * 2
            + [pltpu.VMEM((CHUNK, D_FEAT), jnp.float32)] * NBUF
            + [pltpu.VMEM_SHARED((NPAD, D_FEAT), jnp.float32)]
            + [pltpu.SemaphoreType.DMA] * (2 * NIDX + NBUF + NBUF)
        ),
    )
    def spmm_kernel(x_hbm, ei_hbm, tail_hbm, z_hbm, part_hbm, isrc, idst,
                    *rest):
        rows = rest[:NBUF]
        acc = rest[NBUF]
        semis = rest[NBUF + 1:NBUF + 1 + NIDX]
        semid = rest[NBUF + 1 + NIDX:NBUF + 1 + 2 * NIDX]
        semg = rest[NBUF + 1 + 2 * NIDX:NBUF + 1 + 2 * NIDX + NBUF]
        sems = rest[NBUF + 1 + 2 * NIDX + NBUF:]

        c = lax.axis_index("c")
        s = lax.axis_index("s")
        wid = c * NS + s

        def idx_start(j, b):
            def main(jj):
                off = wid * EPT + jj * CHUNK
                pltpu.async_copy(ei_hbm.at[pl.ds(off, CHUNK)], isrc.at[b],
                                 semis[b])
                pltpu.async_copy(ei_hbm.at[pl.ds(N_EDGES + off, CHUNK)],
                                 idst.at[b], semid[b])

            def tail_chunk():
                pltpu.async_copy(tail_hbm.at[pl.ds(wid * CHUNK, CHUNK)],
                                 isrc.at[b], semis[b])
                pltpu.async_copy(
                    tail_hbm.at[pl.ds((NW + wid) * CHUNK, CHUNK)],
                    idst.at[b], semid[b])

            if isinstance(j, int):
                if j < CPWM:
                    main(j)
                else:
                    tail_chunk()
            else:
                @pl.when(j < CPWM)
                def _():
                    main(j)

                @pl.when(j == CPWM)
                def _():
                    tail_chunk()

        def idx_wait(j, b):
            pltpu.make_async_copy(tail_hbm.at[pl.ds(0, CHUNK)], isrc.at[b],
                                  semis[b]).wait()
            pltpu.make_async_copy(tail_hbm.at[pl.ds(0, CHUNK)], idst.at[b],
                                  semid[b]).wait()

        def gather_start(b, q):
            pltpu.async_copy(x_hbm.at[isrc.at[b]], rows[q], semg[q])

        def gather_wait(b, q):
            pltpu.make_async_copy(x_hbm.at[isrc.at[b]], rows[q],
                                  semg[q]).wait()

        def scat_start(b, q):
            pltpu.async_copy(rows[q], acc.at[idst.at[b]], sems[q],
                             add=True)

        def scat_wait(b, q):
            pltpu.make_async_copy(rows[q], acc.at[idst.at[b]],
                                  sems[q]).wait()

        pltpu.sync_copy(z_hbm, rows[0])
        base0 = s * RPW
        for k in range(RPW // CHUNK):
            pltpu.sync_copy(rows[0], acc.at[pl.ds(base0 + k * CHUNK, CHUNK)])
        rem = RPW - (RPW // CHUNK) * CHUNK
        if rem:
            pltpu.sync_copy(rows[0].at[pl.ds(0, rem)],
                            acc.at[pl.ds(base0 + RPW - rem, rem)])
        plsc.subcore_barrier()

        for t in range(4):
            idx_start(t, t)
        idx_wait(0, 0)
        gather_start(0, 0)

        def body(i, carry):
            base = 6 * i
            for u in range(6):
                j = base + u
                q = u % NBUF
                q1 = (u + 1) % NBUF
                b1 = (u + 1) % NIDX
                b4 = (u + 4) % NIDX
                bm2 = (u - 2) % NIDX

                @pl.when(j >= 2)
                def _():
                    scat_wait(bm2, q1)

                @pl.when(j + 4 < CPW)
                def _():
                    idx_start(j + 4, b4)

                @pl.when(j + 1 < CPW)
                def _():
                    idx_wait(j + 1, b1)
                    gather_start(b1, q1)

                gather_wait(u, q)
                scat_start(u, q)
            return carry
        lax.fori_loop(0, CPW // 6, body, 0)

        for t in (CPW - 2, CPW - 1):
            scat_wait(t % NIDX, t % NBUF)
        plsc.subcore_barrier()

        pltpu.sync_copy(acc.at[pl.ds(s * RPW, RPW)],
                        part_hbm.at[c, pl.ds(s * RPW, RPW)])

    return spmm_kernel(x, ei, tail, zeros_blk)


def _tc_add(part):
    def add_kernel(a_ref, b_ref, o_ref):
        o_ref[...] = a_ref[...] + b_ref[...]

    return pl.pallas_call(
        add_kernel,
        grid=(10,),
        in_specs=[
            pl.BlockSpec((None, 1000, D_FEAT), lambda i: (0, i, 0)),
            pl.BlockSpec((None, 1000, D_FEAT), lambda i: (1, i, 0)),
        ],
        out_specs=pl.BlockSpec((1000, D_FEAT), lambda i: (i, 0)),
        out_shape=jax.ShapeDtypeStruct((N_NODES, D_FEAT), jnp.float32),
    )(part, part)


def kernel(x, edge_index):
    ei = edge_index.astype(jnp.int32)
    t_real = ei.reshape(2, NW, EPT)[:, :, CPWM * CHUNK:]
    ei = ei.reshape(2 * N_EDGES)
    pad_src = (jnp.arange(NW * TPAD, dtype=jnp.int32)
               % N_NODES).reshape(NW, TPAD)
    pad_dst = (N_NODES + jnp.arange(NW * TPAD, dtype=jnp.int32)
               % (NPAD - N_NODES)).reshape(NW, TPAD)
    tail = jnp.stack([
        jnp.concatenate([t_real[0], pad_src], axis=1),
        jnp.concatenate([t_real[1], pad_dst], axis=1),
    ]).reshape(2 * NW * CHUNK)
    zeros_blk = jnp.zeros((CHUNK, D_FEAT), jnp.float32)
    part = _sc_spmm(x, ei, tail, zeros_blk)
    return _tc_add(part)


# speedup vs baseline: 15.2685x; 1.0786x over previous
import functools

import jax
import jax.numpy as jnp
from jax import lax
from jax.experimental import pallas as pl
from jax.experimental.pallas import tpu as pltpu
from jax.experimental.pallas import tpu_sc as plsc

N_NODES = 10000
D_FEAT = 128
N_EDGES = 320000

NC = 2
NS = 16
NW = NC * NS
CHUNK = 128
NCHUNKS = N_EDGES // CHUNK
CPW = 79
NBUF = 3
NIDX = 4
NPAD = 10104
RPW = 632


def _sc_spmm(x, ei, zeros_blk):
    mesh = plsc.VectorSubcoreMesh(core_axis_name="c", subcore_axis_name="s")

    @functools.partial(
        pl.kernel,
        out_type=jax.ShapeDtypeStruct((NC, NPAD, D_FEAT), jnp.float32),
        mesh=mesh,
        scratch_types=(
            [pltpu.VMEM((NIDX, 2, CHUNK), jnp.int32)]
            + [pltpu.VMEM((CHUNK, D_FEAT), jnp.float32)] * NBUF
            + [pltpu.VMEM_SHARED((NPAD, D_FEAT), jnp.float32)]
            + [pltpu.SemaphoreType.DMA] * (NIDX + NBUF + NBUF)
        ),
    )
    def spmm_kernel(x_hbm, ei_hbm, z_hbm, part_hbm, ibuf, *rest):
        rows = rest[:NBUF]
        acc = rest[NBUF]
        semi = rest[NBUF + 1:NBUF + 1 + NIDX]
        semg = rest[NBUF + 1 + NIDX:NBUF + 1 + NIDX + NBUF]
        sems = rest[NBUF + 1 + NIDX + NBUF:]

        c = lax.axis_index("c")
        s = lax.axis_index("s")
        wid = c * NS + s
        chunk0 = (625 * wid) // 8
        chunk1 = (625 * (wid + 1)) // 8
        short = (chunk1 - chunk0) == (CPW - 1)

        def idx_start(j, b):
            k = chunk0 + j
            if not isinstance(j, int):
                k = jnp.where(short & (j == CPW - 1), chunk0, k)
            pltpu.async_copy(ei_hbm.at[:, pl.ds(k * CHUNK, CHUNK)],
                             ibuf.at[b], semi[b])

        def idx_wait(j, b):
            pltpu.make_async_copy(ei_hbm.at[:, pl.ds(0, CHUNK)], ibuf.at[b],
                                  semi[b]).wait()

        def gather_start(j, b, q):
            if isinstance(j, int):
                pltpu.async_copy(x_hbm.at[ibuf.at[b, 0]], rows[q], semg[q])
            else:
                ghost = short & (j == CPW - 1)

                @pl.when(jnp.logical_not(ghost))
                def _():
                    pltpu.async_copy(x_hbm.at[ibuf.at[b, 0]], rows[q],
                                     semg[q])

                @pl.when(ghost)
                def _():
                    pltpu.async_copy(z_hbm, rows[q], semg[q])

        def gather_wait(b, q):
            pltpu.make_async_copy(z_hbm, rows[q], semg[q]).wait()

        def scat_start(b, q):
            pltpu.async_copy(rows[q], acc.at[ibuf.at[b, 1]], sems[q],
                             add=True)

        def scat_wait(b, q):
            pltpu.make_async_copy(rows[q], acc.at[ibuf.at[b, 1]],
                                  sems[q]).wait()

        pltpu.sync_copy(z_hbm, rows[0])
        base0 = s * RPW
        for k in range(4):
            pltpu.sync_copy(rows[0], acc.at[pl.ds(base0 + k * CHUNK, CHUNK)])

        @pl.when(s < NS - 1)
        def _():
            pltpu.sync_copy(rows[0].at[pl.ds(0, RPW - 4 * CHUNK)],
                            acc.at[pl.ds(base0 + 4 * CHUNK,
                                         RPW - 4 * CHUNK)])

        @pl.when(s == NS - 1)
        def _():
            pltpu.sync_copy(rows[0].at[pl.ds(0, RPW - 8 - 4 * CHUNK)],
                            acc.at[pl.ds(base0 + 4 * CHUNK,
                                         RPW - 8 - 4 * CHUNK)])
        plsc.subcore_barrier()

        idx_start(0, 0)
        idx_start(1, 1)
        idx_wait(0, 0)
        gather_start(0, 0, 0)

        def stage(j, u):
            q = u % NBUF
            q1 = (u + 1) % NBUF
            b = u % NIDX
            b1 = (u + 1) % NIDX
            b2 = (u + 2) % NIDX

            @pl.when(j >= 2)
            def _():
                scat_wait(b2, q1)

            @pl.when(j + 2 < CPW)
            def _():
                idx_start(j + 2, b2)

            @pl.when(j + 1 < CPW)
            def _():
                idx_wait(j + 1, b1)
                gather_start(j + 1, b1, q1)

            gather_wait(b, q)
            scat_start(b, q)

        def body(i, carry):
            base = 12 * i
            for u in range(12):
                stage(base + u, u)
            return carry
        lax.fori_loop(0, 6, body, 0)
        for t in range(72, CPW):
            stage(jnp.int32(t), t % 12)

        for t in (CPW - 2, CPW - 1):
            scat_wait(t % NIDX, t % NBUF)
        plsc.subcore_barrier()

        @pl.when(s < NS - 1)
        def _():
            pltpu.sync_copy(acc.at[pl.ds(s * RPW, RPW)],
                            part_hbm.at[c, pl.ds(s * RPW, RPW)])

        @pl.when(s == NS - 1)
        def _():
            pltpu.sync_copy(acc.at[pl.ds((NS - 1) * RPW, RPW - 8)],
                            part_hbm.at[c, pl.ds((NS - 1) * RPW, RPW - 8)])

    return spmm_kernel(x, ei, zeros_blk)


def _tc_add(part):
    def add_kernel(a_ref, b_ref, o_ref):
        o_ref[...] = a_ref[...] + b_ref[...]

    return pl.pallas_call(
        add_kernel,
        grid=(10,),
        in_specs=[
            pl.BlockSpec((None, 1000, D_FEAT), lambda i: (0, i, 0)),
            pl.BlockSpec((None, 1000, D_FEAT), lambda i: (1, i, 0)),
        ],
        out_specs=pl.BlockSpec((1000, D_FEAT), lambda i: (i, 0)),
        out_shape=jax.ShapeDtypeStruct((N_NODES, D_FEAT), jnp.float32),
    )(part, part)


def kernel(x, edge_index):
    ei = edge_index.astype(jnp.int32)
    zeros_blk = jnp.zeros((CHUNK, D_FEAT), jnp.float32)
    part = _sc_spmm(x, ei, zeros_blk)
    return _tc_add(part)


# speedup vs baseline: 15.5550x; 1.0188x over previous
import functools

import jax
import jax.numpy as jnp
from jax import lax
from jax.experimental import pallas as pl
from jax.experimental.pallas import tpu as pltpu
from jax.experimental.pallas import tpu_sc as plsc

N_NODES = 10000
D_FEAT = 128
N_EDGES = 320000

NC = 2
NS = 16
NW = NC * NS
CHUNK = 128
NCHUNKS = N_EDGES // CHUNK
CPW = 79
NBUF = 3
NIDX = 4
NPAD = 10104
RPW = 632


def _sc_spmm(x, ei, zeros_blk):
    mesh = plsc.VectorSubcoreMesh(core_axis_name="c", subcore_axis_name="s")

    @functools.partial(
        pl.kernel,
        out_type=jax.ShapeDtypeStruct((NC, NPAD, D_FEAT), jnp.float32),
        mesh=mesh,
        scratch_types=(
            [pltpu.VMEM((NIDX, 2, CHUNK), jnp.int32)]
            + [pltpu.VMEM((CHUNK, D_FEAT), jnp.float32)] * NBUF
            + [pltpu.VMEM_SHARED((NPAD, D_FEAT), jnp.float32)]
            + [pltpu.SemaphoreType.DMA] * (NIDX + NBUF + NBUF)
        ),
    )
    def spmm_kernel(x_hbm, ei_hbm, z_hbm, part_hbm, ibuf, *rest):
        rows = rest[:NBUF]
        acc = rest[NBUF]
        semi = rest[NBUF + 1:NBUF + 1 + NIDX]
        semg = rest[NBUF + 1 + NIDX:NBUF + 1 + NIDX + NBUF]
        sems = rest[NBUF + 1 + NIDX + NBUF:]

        c = lax.axis_index("c")
        s = lax.axis_index("s")
        wid = c * NS + s
        chunk0 = (625 * wid) // 8
        chunk1 = (625 * (wid + 1)) // 8
        short = (chunk1 - chunk0) == (CPW - 1)

        def idx_start(j, b):
            k = chunk0 + j
            if not isinstance(j, int):
                k = jnp.where(short & (j == CPW - 1), chunk0, k)
            pltpu.async_copy(ei_hbm.at[:, pl.ds(k * CHUNK, CHUNK)],
                             ibuf.at[b], semi[b])

        def idx_wait(j, b):
            pltpu.make_async_copy(ei_hbm.at[:, pl.ds(0, CHUNK)], ibuf.at[b],
                                  semi[b]).wait()

        def gather_start(j, b, q):
            if isinstance(j, int):
                pltpu.async_copy(x_hbm.at[ibuf.at[b, 0]], rows[q], semg[q])
            else:
                ghost = short & (j == CPW - 1)

                @pl.when(jnp.logical_not(ghost))
                def _():
                    pltpu.async_copy(x_hbm.at[ibuf.at[b, 0]], rows[q],
                                     semg[q])

                @pl.when(ghost)
                def _():
                    pltpu.async_copy(z_hbm, rows[q], semg[q])

        def gather_wait(b, q):
            pltpu.make_async_copy(z_hbm, rows[q], semg[q]).wait()

        def scat_start(b, q):
            pltpu.async_copy(rows[q], acc.at[ibuf.at[b, 1]], sems[q],
                             add=True)

        def scat_wait(b, q):
            pltpu.make_async_copy(rows[q], acc.at[ibuf.at[b, 1]],
                                  sems[q]).wait()

        pltpu.sync_copy(z_hbm, rows[0])
        base0 = s * RPW
        for k in range(4):
            pltpu.sync_copy(rows[0], acc.at[pl.ds(base0 + k * CHUNK, CHUNK)])

        @pl.when(s < NS - 1)
        def _():
            pltpu.sync_copy(rows[0].at[pl.ds(0, RPW - 4 * CHUNK)],
                            acc.at[pl.ds(base0 + 4 * CHUNK,
                                         RPW - 4 * CHUNK)])

        @pl.when(s == NS - 1)
        def _():
            pltpu.sync_copy(rows[0].at[pl.ds(0, RPW - 8 - 4 * CHUNK)],
                            acc.at[pl.ds(base0 + 4 * CHUNK,
                                         RPW - 8 - 4 * CHUNK)])
        plsc.subcore_barrier()

        idx_start(0, 0)
        idx_start(1, 1)
        idx_wait(0, 0)
        gather_start(0, 0, 0)

        def stage(j, u):
            q = u % NBUF
            q1 = (u + 1) % NBUF
            b = u % NIDX
            b1 = (u + 1) % NIDX
            b2 = (u + 2) % NIDX

            @pl.when(j >= 2)
            def _():
                scat_wait(b2, q1)

            @pl.when(j + 2 < CPW)
            def _():
                idx_start(j + 2, b2)

            @pl.when(j + 1 < CPW)
            def _():
                idx_wait(j + 1, b1)
                gather_start(j + 1, b1, q1)

            gather_wait(b, q)
            scat_start(b, q)

        def body(i, carry):
            base = 12 * i
            for u in range(12):
                stage(base + u, u)
            return carry
        lax.fori_loop(0, 6, body, 0)
        for t in range(72, CPW):
            stage(jnp.int32(t), t % 12)

        for t in (CPW - 2, CPW - 1):
            scat_wait(t % NIDX, t % NBUF)
        plsc.subcore_barrier()

        @pl.when(s < NS - 1)
        def _():
            pltpu.sync_copy(acc.at[pl.ds(s * RPW, RPW)],
                            part_hbm.at[c, pl.ds(s * RPW, RPW)])

        @pl.when(s == NS - 1)
        def _():
            pltpu.sync_copy(acc.at[pl.ds((NS - 1) * RPW, RPW - 8)],
                            part_hbm.at[c, pl.ds((NS - 1) * RPW, RPW - 8)])

    return spmm_kernel(x, ei, zeros_blk)


def _tc_add(part):
    def add_kernel(a_ref, b_ref, o_ref):
        o_ref[...] = a_ref[...] + b_ref[...]

    return pl.pallas_call(
        add_kernel,
        grid=(5,),
        in_specs=[
            pl.BlockSpec((None, 2000, D_FEAT), lambda i: (0, i, 0)),
            pl.BlockSpec((None, 2000, D_FEAT), lambda i: (1, i, 0)),
        ],
        out_specs=pl.BlockSpec((2000, D_FEAT), lambda i: (i, 0)),
        out_shape=jax.ShapeDtypeStruct((N_NODES, D_FEAT), jnp.float32),
    )(part, part)


def kernel(x, edge_index):
    ei = edge_index.astype(jnp.int32)
    zeros_blk = jnp.zeros((CHUNK, D_FEAT), jnp.float32)
    part = _sc_spmm(x, ei, zeros_blk)
    return _tc_add(part)
